# baseline (speedup 1.0000x reference)
import functools

import jax
import jax.numpy as jnp
from jax import lax
from jax.experimental import pallas as pl
from jax.experimental.pallas import tpu as pltpu
from jax.experimental.pallas import tpu_sc as plsc

N_NODES = 10000
N_EDGES = 320000
D_FEAT = 128

NUM_CORES = 2
NUM_SUBCORES = 16
NUM_WORKERS = NUM_CORES * NUM_SUBCORES

BLK = 64
NBLOCKS = N_EDGES // BLK
NB = 160
NB_LAST = NBLOCKS - (NUM_WORKERS - 1) * NB
N_PAD = NUM_SUBCORES * 640
ZROWS = 640
NBUF = 4
NB_CHUNK = 80


def _sc_body(x_hbm, srcs_hbm, dsts_hbm, out_hbm,
             src_v, dst_v, rows_v, acc_sh, sem, ssem):
  c = lax.axis_index("c")
  s = lax.axis_index("s")
  w = c * NUM_SUBCORES + s
  is_last = w == NUM_WORKERS - 1

  zv = jnp.zeros((16,), jnp.float32)

  def zstep(i, carry):
    for j in range(8):
      rows_v[0, i, pl.ds(j * 16, 16)] = zv
    return carry

  lax.fori_loop(0, BLK, zstep, 0)
  for r in range(ZROWS // BLK):
    pltpu.sync_copy(rows_v.at[0],
                    acc_sh.at[pl.ds(s * ZROWS + r * BLK, BLK)])
  plsc.subcore_barrier()

  def gather(b, k):
    return pltpu.make_async_copy(x_hbm.at[src_v.at[b]], rows_v.at[k],
                                 sem.at[k])

  def scat_wait(b, k):
    pltpu.make_async_copy(rows_v.at[k], acc_sh.at[dst_v.at[b]],
                          ssem.at[k]).wait()

  def run_chunk(start_blk, n_blk):
    pltpu.sync_copy(srcs_hbm.at[pl.ds(start_blk, n_blk)],
                    src_v.at[pl.ds(0, n_blk)])
    pltpu.sync_copy(dsts_hbm.at[pl.ds(start_blk, n_blk)],
                    dst_v.at[pl.ds(0, n_blk)])

    for k in range(NBUF):
      gather(k, k).start()

    def step(i, carry):
      for k in range(NBUF):
        b = i * NBUF + k
        gather(b, k).wait()
        pltpu.async_copy(rows_v.at[k], acc_sh.at[dst_v.at[b]],
                         ssem.at[k], add=True)
        bp = b - NBUF // 2
        kp = (k + NBUF // 2) % NBUF

        @pl.when((bp >= 0) & (bp + NBUF < n_blk))
        def _():
          scat_wait(bp, kp)
          gather(bp + NBUF, kp).start()
      return carry

    lax.fori_loop(0, n_blk // NBUF, step, 0)
    for k in range(NBUF):
      scat_wait(n_blk - NBUF + k, k)

  base = pl.multiple_of(w * NB, 8)

  @pl.when(jnp.logical_not(is_last))
  def _():
    for h in range(NB // NB_CHUNK):
      run_chunk(base + h * NB_CHUNK, NB_CHUNK)

  @pl.when(is_last)
  def _():
    run_chunk(base, NB_LAST)

  plsc.subcore_barrier()
  pltpu.sync_copy(acc_sh.at[pl.ds(s * ZROWS, ZROWS)],
                  out_hbm.at[c].at[pl.ds(s * ZROWS, ZROWS)])


@functools.partial(
    pl.kernel,
    out_type=jax.ShapeDtypeStruct((NUM_CORES, N_PAD, D_FEAT), jnp.float32),
    mesh=plsc.VectorSubcoreMesh(core_axis_name="c", subcore_axis_name="s"),
    scratch_types=[
        pltpu.VMEM((NB_CHUNK, BLK), jnp.int32),
        pltpu.VMEM((NB_CHUNK, BLK), jnp.int32),
        pltpu.VMEM((NBUF, BLK, D_FEAT), jnp.float32),
        pltpu.VMEM_SHARED((N_PAD, D_FEAT), jnp.float32),
        pltpu.SemaphoreType.DMA((NBUF,)),
        pltpu.SemaphoreType.DMA((NBUF,)),
    ],
    compiler_params=pltpu.CompilerParams(use_tc_tiling_on_sc=False),
)
def _mp_scatter_kernel(x_hbm, srcs_hbm, dsts_hbm, out_hbm,
                       src_v, dst_v, rows_v, acc_sh, sem, ssem):
  _sc_body(x_hbm, srcs_hbm, dsts_hbm, out_hbm,
           src_v, dst_v, rows_v, acc_sh, sem, ssem)


def _combine_body(a_ref, b_ref, o_ref):
  o_ref[...] = a_ref[...] + b_ref[...]


def _combine(partials):
  blk = 1000
  return pl.pallas_call(
      _combine_body,
      grid=(N_NODES // blk,),
      in_specs=[
          pl.BlockSpec((blk, D_FEAT), lambda i: (i, 0)),
          pl.BlockSpec((blk, D_FEAT), lambda i: (i, 0)),
      ],
      out_specs=pl.BlockSpec((blk, D_FEAT), lambda i: (i, 0)),
      out_shape=jax.ShapeDtypeStruct((N_NODES, D_FEAT), jnp.float32),
  )(partials[0], partials[1])


@jax.jit
def kernel(x, edge_index):
  srcs = edge_index[0].astype(jnp.int32).reshape(NBLOCKS, BLK)
  dsts = edge_index[1].astype(jnp.int32).reshape(NBLOCKS, BLK)
  partials = _mp_scatter_kernel(x, srcs, dsts)
  return _combine(partials)


# speedup vs baseline: 1.1391x; 1.1391x over previous
import functools

import jax
import jax.numpy as jnp
from jax import lax
from jax.experimental import pallas as pl
from jax.experimental.pallas import tpu as pltpu
from jax.experimental.pallas import tpu_sc as plsc

N_NODES = 10000
N_EDGES = 320000
D_FEAT = 128

NUM_CORES = 2
NUM_SUBCORES = 16
NUM_WORKERS = NUM_CORES * NUM_SUBCORES

BLK = 128
NBLOCKS = N_EDGES // BLK
NB = 80
NB_LAST = NBLOCKS - (NUM_WORKERS - 1) * NB
N_PAD = NUM_SUBCORES * 640
ZROWS = 640
NBUF = 2
NB_CHUNK = 40


def _sc_body(x_hbm, edges_hbm, out_hbm, src_v, dst_v, rows_v, acc_sh, sem):
  c = lax.axis_index("c")
  s = lax.axis_index("s")
  w = c * NUM_SUBCORES + s
  is_last = w == NUM_WORKERS - 1
  srcs_hbm = edges_hbm.at[0]
  dsts_hbm = edges_hbm.at[1]

  zv = jnp.zeros((16,), jnp.float32)

  def zstep(i, carry):
    for j in range(8):
      rows_v[0, i, pl.ds(j * 16, 16)] = zv
    return carry

  lax.fori_loop(0, BLK, zstep, 0)
  for r in range(ZROWS // BLK):
    pltpu.sync_copy(rows_v.at[0],
                    acc_sh.at[pl.ds(s * ZROWS + r * BLK, BLK)])
  plsc.subcore_barrier()

  def gather_half(b, k, half):
    return pltpu.make_async_copy(
        x_hbm.at[src_v.at[b].at[pl.ds(64 * half, 64)]],
        rows_v.at[k].at[pl.ds(64 * half, 64)],
        sem.at[2 * k + half])

  def gather_start(b, k):
    gather_half(b, k, 0).start()
    gather_half(b, k, 1).start()

  def gather_wait(b, k):
    gather_half(b, k, 0).wait()
    gather_half(b, k, 1).wait()

  def run_chunk(start_blk, n_blk):
    pltpu.sync_copy(srcs_hbm.at[pl.ds(start_blk, n_blk)],
                    src_v.at[pl.ds(0, n_blk)])
    pltpu.sync_copy(dsts_hbm.at[pl.ds(start_blk, n_blk)],
                    dst_v.at[pl.ds(0, n_blk)])

    for k in range(NBUF):
      gather_start(k, k)

    def step(i, carry):
      for k in range(NBUF):
        b = i * NBUF + k
        gather_wait(b, k)
        pltpu.sync_copy(rows_v.at[k], acc_sh.at[dst_v.at[b]], add=True)

        @pl.when(b + NBUF < n_blk)
        def _():
          gather_start(b + NBUF, k)
      return carry

    lax.fori_loop(0, n_blk // NBUF, step, 0)

  base = pl.multiple_of(w * NB, 8)

  @pl.when(jnp.logical_not(is_last))
  def _():
    for h in range(NB // NB_CHUNK):
      run_chunk(base + h * NB_CHUNK, NB_CHUNK)

  @pl.when(is_last)
  def _():
    run_chunk(base, NB_LAST)

  plsc.subcore_barrier()
  pltpu.sync_copy(acc_sh.at[pl.ds(s * ZROWS, ZROWS)],
                  out_hbm.at[c].at[pl.ds(s * ZROWS, ZROWS)])


@functools.partial(
    pl.kernel,
    out_type=jax.ShapeDtypeStruct((NUM_CORES, N_PAD, D_FEAT), jnp.float32),
    mesh=plsc.VectorSubcoreMesh(core_axis_name="c", subcore_axis_name="s"),
    scratch_types=[
        pltpu.VMEM((NB_CHUNK, BLK), jnp.int32),
        pltpu.VMEM((NB_CHUNK, BLK), jnp.int32),
        pltpu.VMEM((NBUF, BLK, D_FEAT), jnp.float32),
        pltpu.VMEM_SHARED((N_PAD, D_FEAT), jnp.float32),
        pltpu.SemaphoreType.DMA((2 * NBUF,)),
    ],
    compiler_params=pltpu.CompilerParams(use_tc_tiling_on_sc=False),
)
def _mp_scatter_kernel(x_hbm, edges_hbm, out_hbm,
                       src_v, dst_v, rows_v, acc_sh, sem):
  _sc_body(x_hbm, edges_hbm, out_hbm, src_v, dst_v, rows_v, acc_sh, sem)


def _combine_body(a_ref, b_ref, o_ref):
  o_ref[...] = a_ref[...] + b_ref[...]


def _combine(partials):
  blk = 1000
  return pl.pallas_call(
      _combine_body,
      grid=(N_NODES // blk,),
      in_specs=[
          pl.BlockSpec((blk, D_FEAT), lambda i: (i, 0)),
          pl.BlockSpec((blk, D_FEAT), lambda i: (i, 0)),
      ],
      out_specs=pl.BlockSpec((blk, D_FEAT), lambda i: (i, 0)),
      out_shape=jax.ShapeDtypeStruct((N_NODES, D_FEAT), jnp.float32),
  )(partials[0], partials[1])


@jax.jit
def kernel(x, edge_index):
  edges = edge_index.astype(jnp.int32).reshape(2, NBLOCKS, BLK)
  partials = _mp_scatter_kernel(x, edges)
  return _combine(partials)
